# Initial kernel scaffold; baseline (speedup 1.0000x reference)
import jax
import jax.numpy as jnp
from jax.experimental import pallas as pl


def kernel(x, poly_conn, poly_index, Q_w, Q_b, K_w, K_b, E_w, conn_w, conn_b):
    raise NotImplementedError("write your pallas kernel here")


# speedup vs baseline: 2.9889x; 2.9889x over previous
import functools

import jax
import jax.numpy as jnp
from jax import lax
from jax.experimental import pallas as pl
from jax.experimental.pallas import tpu as pltpu
from jax.experimental.pallas import tpu_sc as plsc

N = 10000
E = 320000
D = 128
A = 128

_info = plsc.get_sparse_core_info()
NC = _info.num_cores
NS = _info.num_subcores
NW = NC * NS
EPW = E // NW
C = 80
NCHUNK = EPW // C
NPAD = 10240
SPT = NPAD // NS
ZR = 128


def _qk_body(x_ref, qwt_ref, kwt_ref, qb_ref, kb_ref, qo_ref, ko_ref):
    xv = x_ref[...]
    qo_ref[...] = jnp.dot(xv, qwt_ref[...],
                          preferred_element_type=jnp.float32) + qb_ref[...]
    ko_ref[...] = jnp.dot(xv, kwt_ref[...],
                          preferred_element_type=jnp.float32) + kb_ref[...]


def _qk_proj(x, qwt, kwt, qb, kb):
    BN = 1000
    grid = (N // BN,)
    return pl.pallas_call(
        _qk_body,
        grid=grid,
        in_specs=[
            pl.BlockSpec((BN, D), lambda i: (i, 0)),
            pl.BlockSpec((D, A), lambda i: (0, 0)),
            pl.BlockSpec((D, A), lambda i: (0, 0)),
            pl.BlockSpec((1, A), lambda i: (0, 0)),
            pl.BlockSpec((1, A), lambda i: (0, 0)),
        ],
        out_specs=[
            pl.BlockSpec((BN, A), lambda i: (i, 0)),
            pl.BlockSpec((BN, A), lambda i: (i, 0)),
        ],
        out_shape=[
            jax.ShapeDtypeStruct((N, A), jnp.float32),
            jax.ShapeDtypeStruct((N, A), jnp.float32),
        ],
    )(x, qwt, kwt, qb, kb)


def _gather_body(qx_hbm, kx_hbm, dst_hbm, src_hbm, out_hbm,
                 didx, sidx, bufq, bufk, semq, semk):
    wid = lax.axis_index("s") * NC + lax.axis_index("c")
    base = wid * EPW

    def chunk(c, carry):
        off = base + c * C
        pltpu.sync_copy(dst_hbm.at[pl.ds(off, C)], didx)
        pltpu.sync_copy(src_hbm.at[pl.ds(off, C)], sidx)
        cq = pltpu.async_copy(qx_hbm.at[didx], bufq, semq)
        ck = pltpu.async_copy(kx_hbm.at[sidx], bufk, semk)
        cq.wait()
        ck.wait()

        def row(r, rc):
            for j in range(A // 16):
                sl = pl.ds(j * 16, 16)
                bufq[r, sl] = bufq[r, sl] + bufk[r, sl]
            return rc

        lax.fori_loop(0, C, row, 0)
        pltpu.sync_copy(bufq, out_hbm.at[pl.ds(off, C)])
        return carry

    lax.fori_loop(0, NCHUNK, chunk, 0)


def _gather_add(qx, kx, dst, src):
    mesh = plsc.VectorSubcoreMesh(core_axis_name="c", subcore_axis_name="s")
    return pl.kernel(
        _gather_body,
        mesh=mesh,
        out_type=jax.ShapeDtypeStruct((E, A), jnp.float32),
        scratch_types=[
            pltpu.VMEM((C,), jnp.int32),
            pltpu.VMEM((C,), jnp.int32),
            pltpu.VMEM((C, A), jnp.float32),
            pltpu.VMEM((C, A), jnp.float32),
            pltpu.SemaphoreType.DMA,
            pltpu.SemaphoreType.DMA,
        ],
    )(qx, kx, dst, src)


def _mlp_body(poly_ref, s_ref, ewt_ref, cwt_ref, cb_ref, out_ref):
    h = s_ref[...] + jnp.dot(poly_ref[...], ewt_ref[...],
                             preferred_element_type=jnp.float32)
    h = jnp.maximum(h, 0.0)
    out_ref[...] = jnp.dot(h, cwt_ref[...],
                           preferred_element_type=jnp.float32) + cb_ref[...]


def _edge_mlp(poly_conn, s, ewt, cwt, cb):
    BE = 2000
    grid = (E // BE,)
    return pl.pallas_call(
        _mlp_body,
        grid=grid,
        in_specs=[
            pl.BlockSpec((BE, D), lambda i: (i, 0)),
            pl.BlockSpec((BE, A), lambda i: (i, 0)),
            pl.BlockSpec((D, A), lambda i: (0, 0)),
            pl.BlockSpec((A, D), lambda i: (0, 0)),
            pl.BlockSpec((1, D), lambda i: (0, 0)),
        ],
        out_specs=pl.BlockSpec((BE, D), lambda i: (i, 0)),
        out_shape=jax.ShapeDtypeStruct((E, D), jnp.float32),
    )(poly_conn, s, ewt, cwt, cb)


def _scatter_body(conn_hbm, dstm_hbm, out_hbm, idxbuf, cbuf, zbuf, agg_sh, sem):
    cid = lax.axis_index("c")
    sid = lax.axis_index("s")
    wid = sid * NC + cid

    zero = jnp.zeros((16,), jnp.float32)

    def zrow(r, rc):
        for j in range(D // 16):
            zbuf[r, pl.ds(j * 16, 16)] = zero
        return rc

    lax.fori_loop(0, ZR, zrow, 0)

    def zcp(i, ic):
        pltpu.sync_copy(zbuf, agg_sh.at[pl.ds(sid * SPT + i * ZR, ZR)])
        return ic

    lax.fori_loop(0, SPT // ZR, zcp, 0)
    plsc.subcore_barrier()

    pltpu.sync_copy(dstm_hbm.at[wid], idxbuf)

    def chunk(ch, cc):
        off = (wid * NCHUNK + ch) * C
        pltpu.sync_copy(conn_hbm.at[pl.ds(off, C)], cbuf)
        pltpu.sync_copy(cbuf, agg_sh.at[idxbuf.at[ch]], add=True)
        return cc

    lax.fori_loop(0, NCHUNK, chunk, 0)
    plsc.subcore_barrier()

    def wcp(i, ic):
        r0 = sid * SPT + i * ZR
        pltpu.sync_copy(agg_sh.at[pl.ds(r0, ZR)], out_hbm.at[cid].at[pl.ds(r0, ZR)])
        return ic

    lax.fori_loop(0, SPT // ZR, wcp, 0)


def _scatter_add(conn, dstm):
    mesh = plsc.VectorSubcoreMesh(core_axis_name="c", subcore_axis_name="s")
    return pl.kernel(
        _scatter_body,
        mesh=mesh,
        out_type=jax.ShapeDtypeStruct((NC, NPAD, D), jnp.float32),
        scratch_types=[
            pltpu.VMEM((NCHUNK, C), jnp.int32),
            pltpu.VMEM((C, D), jnp.float32),
            pltpu.VMEM((ZR, D), jnp.float32),
            pltpu.VMEM_SHARED((NPAD, D), jnp.float32),
            pltpu.SemaphoreType.DMA,
        ],
    )(conn, dstm)


def _comb_body(p_ref, out_ref):
    out_ref[...] = p_ref[0] + p_ref[1]


def _combine(parts):
    BN = 1000
    return pl.pallas_call(
        _comb_body,
        grid=(N // BN,),
        in_specs=[pl.BlockSpec((NC, BN, D), lambda i: (0, i, 0))],
        out_specs=pl.BlockSpec((BN, D), lambda i: (i, 0)),
        out_shape=jax.ShapeDtypeStruct((N, D), jnp.float32),
    )(parts)


def kernel(x, poly_conn, poly_index, Q_w, Q_b, K_w, K_b, E_w, conn_w, conn_b):
    dst = poly_index[0]
    src = poly_index[1]
    qwt = Q_w.T
    kwt = K_w.T
    ewt = E_w.T
    cwt = conn_w.T
    qb = Q_b.reshape(1, A)
    kb = K_b.reshape(1, A)
    cb = conn_b.reshape(1, D)

    qx, kx = _qk_proj(x, qwt, kwt, qb, kb)
    s = _gather_add(qx, kx, dst, src)
    conn = _edge_mlp(poly_conn, s, ewt, cwt, cb)
    dstm = dst.reshape(NW, NCHUNK, C)
    parts = _scatter_add(conn, dstm)
    agg = _combine(parts)
    return (agg, conn)
